# Initial kernel scaffold; baseline (speedup 1.0000x reference)
import jax
import jax.numpy as jnp
from jax.experimental import pallas as pl


def kernel(latents, vq_weight, codebook):
    raise NotImplementedError("write your pallas kernel here")


# speedup vs baseline: 1.3748x; 1.3748x over previous
import jax
import jax.numpy as jnp
from jax.experimental import pallas as pl

K = 1024
D = 256
NB = 1024


def _vq_block(flat_ref, cb_ref, out_ref, loss_ref):
    flat = flat_ref[...]
    cb = cb_ref[...]
    f2 = jnp.sum(flat * flat, axis=1, keepdims=True)
    cb2 = jnp.sum(cb * cb, axis=1)
    mm = jax.lax.dot_general(flat, cb, (((1,), (1,)), ((), ())),
                             preferred_element_type=jnp.float32)
    dist = (f2 + cb2) - 2.0 * mm
    m = jnp.min(dist, axis=1, keepdims=True)
    iota = jax.lax.broadcasted_iota(jnp.int32, dist.shape, 1)
    idx = jnp.min(jnp.where(dist == m, iota, K), axis=1)
    oh = (iota == idx[:, None]).astype(jnp.float32)
    q = jax.lax.dot_general(oh, cb, (((1,), (0,)), ((), ())),
                            preferred_element_type=jnp.float32)
    diff = q - flat
    out_ref[...] = flat + diff
    loss_ref[...] = jnp.full((1, 1, 128), jnp.sum(diff * diff), jnp.float32)


def kernel(latents, vq_weight, codebook):
    lat = jnp.transpose(latents, (0, 2, 3, 4, 1))
    lat_shape = lat.shape
    flat = lat.reshape(-1, D)
    n = flat.shape[0]
    nblk = n // NB
    out, lossp = pl.pallas_call(
        _vq_block,
        grid=(nblk,),
        in_specs=[pl.BlockSpec((NB, D), lambda i: (i, 0)),
                  pl.BlockSpec((K, D), lambda i: (0, 0))],
        out_specs=[pl.BlockSpec((NB, D), lambda i: (i, 0)),
                   pl.BlockSpec((1, 1, 128), lambda i: (i, 0, 0))],
        out_shape=[jax.ShapeDtypeStruct((n, D), jnp.float32),
                   jax.ShapeDtypeStruct((nblk, 1, 128), jnp.float32)],
    )(flat, codebook)
    s = jnp.sum(lossp[:, 0, 0])
    mean = s / (n * D)
    vq_loss = mean * vq_weight + mean
    out5 = out.reshape(lat_shape)
    return jnp.transpose(out5, (0, 4, 1, 2, 3)), vq_loss
